# Initial kernel scaffold; baseline (speedup 1.0000x reference)
import jax
import jax.numpy as jnp
from jax.experimental import pallas as pl


def kernel(x, token_table, pos_table):
    raise NotImplementedError("write your pallas kernel here")


# speedup vs baseline: 2.5781x; 2.5781x over previous
import functools

import jax
import jax.numpy as jnp
from jax import lax
from jax.experimental import pallas as pl
from jax.experimental.pallas import tpu as pltpu
from jax.experimental.pallas import tpu_sc as plsc

_NUM_WORKERS = 32


@functools.lru_cache(maxsize=None)
def _make_emb_kernel(B, S, D, NB):
    assert B % (_NUM_WORKERS * NB) == 0
    rows_per_w = B // _NUM_WORKERS
    n_chunks = rows_per_w // NB
    half = S // 2
    assert S % 2 == 0 and half <= 128 and D % 16 == 0

    mesh = plsc.VectorSubcoreMesh(core_axis_name="c", subcore_axis_name="s")

    @functools.partial(
        pl.kernel,
        mesh=mesh,
        out_type=jax.ShapeDtypeStruct((B * S, D), jnp.float32),
        scratch_types=[
            pltpu.VMEM((S, D), jnp.float32),
            pltpu.VMEM((NB * 2, half), jnp.int32),
            pltpu.VMEM((NB * S, D), jnp.float32),
            pltpu.SemaphoreType.DMA,
        ],
        compiler_params=pltpu.CompilerParams(use_tc_tiling_on_sc=False),
    )
    def emb(x_hbm, tok_hbm, pos_hbm, out_hbm, pos_v, idx_v, rows_v, sem):
        wid = lax.axis_index("s") * 2 + lax.axis_index("c")
        base = wid * rows_per_w
        pltpu.sync_copy(pos_hbm, pos_v)

        def chunk(ci, carry):
            b0 = base + ci * NB
            pltpu.sync_copy(x_hbm.at[pl.ds(b0 * 2, NB * 2)], idx_v)
            copies = [
                pltpu.async_copy(
                    tok_hbm.at[idx_v.at[t]],
                    rows_v.at[pl.ds(t * half, half)],
                    sem,
                )
                for t in range(NB * 2)
            ]
            for cp in copies:
                cp.wait()

            def add_row(j, c):
                for r in range(NB):
                    for q in range(D // 16):
                        sl = pl.ds(q * 16, 16)
                        rows_v[r * S + j, sl] = rows_v[r * S + j, sl] + pos_v[j, sl]
                return c

            lax.fori_loop(0, S, add_row, 0)
            pltpu.sync_copy(rows_v, out_hbm.at[pl.ds(b0 * S, NB * S)])
            return carry

        lax.fori_loop(0, n_chunks, chunk, 0)

    return emb


def kernel(x, token_table, pos_table):
    B, S = x.shape
    V, D = token_table.shape
    emb = _make_emb_kernel(B, S, D, 4)
    out = emb(x.reshape(B * 2, S // 2).astype(jnp.int32), token_table, pos_table)
    return out.reshape(B, S, D)
